# baseline (speedup 1.0000x reference)
import functools

import jax
import jax.numpy as jnp
from jax import lax
from jax.experimental import pallas as pl
from jax.experimental.pallas import tpu as pltpu
from jax.experimental.pallas import tpu_sc as plsc

NC = 2
NS = 16
CH = 80
RING = 3


def _sc_mesh():
    return plsc.VectorSubcoreMesh(
        core_axis_name="c", subcore_axis_name="s", num_cores=NC, num_subcores=NS
    )


def _make_deg_kernel(n, e, w=128):
    tiles = NC * NS
    ept = e // tiles
    nch = ept // CH
    rpt = (n // NS) // 8 * 8
    rem = n - rpt * NS

    @functools.partial(
        pl.kernel,
        mesh=_sc_mesh(),
        out_type=[
            jax.ShapeDtypeStruct((n, w), jnp.float32),
            jax.ShapeDtypeStruct((n, w), jnp.float32),
        ],
        scratch_types=[
            pltpu.VMEM((CH,), jnp.int32),
            pltpu.VMEM((CH,), jnp.int32),
            pltpu.VMEM((CH, w), jnp.float32),
            pltpu.VMEM_SHARED((n, w), jnp.float32),
            pltpu.SemaphoreType.DMA,
            pltpu.SemaphoreType.DMA,
            pltpu.SemaphoreType.DMA,
            pltpu.SemaphoreType.DMA,
        ],
    )
    def deg_kernel(dst_hbm, ones_hbm, zeros_hbm, out_a, out_b,
                   dstv0, dstv1, onesv, acc, si0, si1, ss0, ss1):
        dstv = [dstv0, dstv1]
        si = [si0, si1]
        ss = [ss0, ss1]
        cid = lax.axis_index("c")
        sid = lax.axis_index("s")
        wid = cid * NS + sid
        row0 = sid * rpt
        pltpu.sync_copy(zeros_hbm.at[pl.ds(0, rpt)], acc.at[pl.ds(row0, rpt)])
        if rem:
            @pl.when(sid == NS - 1)
            def _():
                pltpu.sync_copy(zeros_hbm.at[pl.ds(rpt, rem)],
                                acc.at[pl.ds(NS * rpt, rem)])
        pltpu.sync_copy(ones_hbm, onesv)
        ebase = pl.multiple_of(wid * ept, 8)
        plsc.subcore_barrier()

        def slot(i, b):
            nb = 1 - b

            @pl.when(i >= 2)
            def _():
                pltpu.make_async_copy(onesv, acc.at[dstv[b]], ss[b]).wait()

            @pl.when(i < nch)
            def _():
                off = pl.multiple_of(ebase + i * CH, 8)
                pltpu.async_copy(dst_hbm.at[pl.ds(off, CH)], dstv[b], si[b])

            @pl.when(jnp.logical_and(i >= 1, i <= nch))
            def _():
                pltpu.make_async_copy(dst_hbm.at[pl.ds(ebase, CH)], dstv[nb],
                                      si[nb]).wait()
                pltpu.async_copy(onesv, acc.at[dstv[nb]], ss[nb], add=True)

        pairs = (nch + 2) // 2

        def pair(g, c):
            slot(2 * g, 0)
            slot(2 * g + 1, 1)
            return c

        lax.fori_loop(0, pairs, pair, 0)
        for i in range(max(0, 2 * pairs - 2), nch):
            pltpu.make_async_copy(onesv, acc.at[dstv[i % 2]], ss[i % 2]).wait()
        plsc.subcore_barrier()
        out = [out_a, out_b]
        for c in range(NC):
            @pl.when(cid == c)
            def _(o=out[c]):
                pltpu.sync_copy(acc.at[pl.ds(row0, rpt)], o.at[pl.ds(row0, rpt)])
                if rem:
                    @pl.when(sid == NS - 1)
                    def _():
                        pltpu.sync_copy(acc.at[pl.ds(NS * rpt, rem)],
                                        o.at[pl.ds(NS * rpt, rem)])

    return deg_kernel


def _make_agg_kernel(n, e, d):
    tiles = NC * NS
    ept = e // tiles
    nch = ept // CH
    rpt = (n // NS) // 8 * 8
    rem = n - rpt * NS
    R = RING

    @functools.partial(
        pl.kernel,
        mesh=_sc_mesh(),
        out_type=[
            jax.ShapeDtypeStruct((n, d), jnp.float32),
            jax.ShapeDtypeStruct((n, d), jnp.float32),
        ],
        scratch_types=[pltpu.VMEM((ept,), jnp.int32)]
        + [pltpu.VMEM((CH,), jnp.int32)] * R
        + [pltpu.VMEM((CH, d), jnp.float32)] * R
        + [pltpu.VMEM_SHARED((n, d), jnp.float32)]
        + [pltpu.SemaphoreType.DMA] * (3 * R),
    )
    def agg_kernel(z_hbm, src_hbm, dst_hbm, zeros_hbm, out_a, out_b, *scr):
        srcall = scr[0]
        dstv = list(scr[1:1 + R])
        rows = list(scr[1 + R:1 + 2 * R])
        acc = scr[1 + 2 * R]
        sems = scr[2 + 2 * R:]
        sg = list(sems[0:R])
        si = list(sems[R:2 * R])
        ss = list(sems[2 * R:3 * R])
        cid = lax.axis_index("c")
        sid = lax.axis_index("s")
        wid = cid * NS + sid
        row0 = sid * rpt
        pltpu.sync_copy(zeros_hbm.at[pl.ds(0, rpt)], acc.at[pl.ds(row0, rpt)])
        if rem:
            @pl.when(sid == NS - 1)
            def _():
                pltpu.sync_copy(zeros_hbm.at[pl.ds(rpt, rem)],
                                acc.at[pl.ds(NS * rpt, rem)])
        ebase = pl.multiple_of(wid * ept, 8)
        pltpu.sync_copy(src_hbm.at[pl.ds(ebase, ept)], srcall)
        plsc.subcore_barrier()

        def gidx(i):
            return srcall.at[pl.ds(pl.multiple_of(i * CH, 8), CH)]

        def slot(i, b):
            nb = (b - 1) % R

            @pl.when(i >= R)
            def _():
                pltpu.make_async_copy(rows[b], acc.at[dstv[b]], ss[b]).wait()

            @pl.when(i < nch)
            def _():
                pltpu.async_copy(z_hbm.at[gidx(i)], rows[b], sg[b])
                off = pl.multiple_of(ebase + i * CH, 8)
                pltpu.async_copy(dst_hbm.at[pl.ds(off, CH)], dstv[b], si[b])

            @pl.when(jnp.logical_and(i >= 1, i <= nch))
            def _():
                pltpu.make_async_copy(z_hbm.at[gidx(0)], rows[nb], sg[nb]).wait()
                pltpu.make_async_copy(dst_hbm.at[pl.ds(ebase, CH)], dstv[nb],
                                      si[nb]).wait()
                pltpu.async_copy(rows[nb], acc.at[dstv[nb]], ss[nb], add=True)

        ngrp = (nch + 1 + R - 1) // R
        s_last = ngrp * R - 1

        def grp(g, c):
            for b in range(R):
                slot(R * g + b, b)
            return c

        lax.fori_loop(0, ngrp, grp, 0)
        for i in range(max(0, s_last - R + 1), nch):
            pltpu.make_async_copy(rows[i % R], acc.at[dstv[i % R]],
                                  ss[i % R]).wait()
        plsc.subcore_barrier()
        out = [out_a, out_b]
        for c in range(NC):
            @pl.when(cid == c)
            def _(o=out[c]):
                pltpu.sync_copy(acc.at[pl.ds(row0, rpt)], o.at[pl.ds(row0, rpt)])
                if rem:
                    @pl.when(sid == NS - 1)
                    def _():
                        pltpu.sync_copy(acc.at[pl.ds(NS * rpt, rem)],
                                        o.at[pl.ds(NS * rpt, rem)])

    return agg_kernel


_BM = 1000


def _tc0_body(x_ref, w_ref, xw_ref):
    xw_ref[...] = jnp.dot(x_ref[...], w_ref[...],
                          preferred_element_type=jnp.float32)


def _tc0(x, w1):
    m, din = x.shape
    dh = w1.shape[1]
    grid = (m // _BM,)
    return pl.pallas_call(
        _tc0_body,
        grid=grid,
        in_specs=[
            pl.BlockSpec((_BM, din), lambda i: (i, 0)),
            pl.BlockSpec((din, dh), lambda i: (0, 0)),
        ],
        out_specs=pl.BlockSpec((_BM, dh), lambda i: (i, 0)),
        out_shape=jax.ShapeDtypeStruct((m, dh), jnp.float32),
    )(x, w1)


def _tc1_body(xw_ref, da_ref, db_ref, z_ref, dinv_ref):
    deg = da_ref[:, :1] + db_ref[:, :1] + 1.0
    dinv = lax.rsqrt(deg)
    z_ref[...] = dinv * xw_ref[...]
    dinv_ref[...] = jnp.broadcast_to(dinv, (dinv.shape[0], 16))


def _tc1(xw, deg_a, deg_b):
    m, dh = xw.shape
    grid = (m // _BM,)
    return pl.pallas_call(
        _tc1_body,
        grid=grid,
        in_specs=[
            pl.BlockSpec((_BM, dh), lambda i: (i, 0)),
            pl.BlockSpec((_BM, 128), lambda i: (i, 0)),
            pl.BlockSpec((_BM, 128), lambda i: (i, 0)),
        ],
        out_specs=[
            pl.BlockSpec((_BM, dh), lambda i: (i, 0)),
            pl.BlockSpec((_BM, 16), lambda i: (i, 0)),
        ],
        out_shape=[
            jax.ShapeDtypeStruct((m, dh), jnp.float32),
            jax.ShapeDtypeStruct((m, 16), jnp.float32),
        ],
    )(xw, deg_a, deg_b)


def _tc2_body(aa_ref, ab_ref, z1_ref, dinv_ref, w2_ref, b1_ref, z2_ref):
    dv = dinv_ref[:, :1]
    h = jnp.maximum(dv * (aa_ref[...] + ab_ref[...] + z1_ref[...]) + b1_ref[...], 0.0)
    z2_ref[...] = dv * jnp.dot(h, w2_ref[...], preferred_element_type=jnp.float32)


def _tc2(acc_a, acc_b, z1, dinv16, w2, b1_row):
    m, dh = z1.shape
    grid = (m // _BM,)
    return pl.pallas_call(
        _tc2_body,
        grid=grid,
        in_specs=[
            pl.BlockSpec((_BM, dh), lambda i: (i, 0)),
            pl.BlockSpec((_BM, dh), lambda i: (i, 0)),
            pl.BlockSpec((_BM, dh), lambda i: (i, 0)),
            pl.BlockSpec((_BM, 16), lambda i: (i, 0)),
            pl.BlockSpec((dh, dh), lambda i: (0, 0)),
            pl.BlockSpec((1, dh), lambda i: (0, 0)),
        ],
        out_specs=pl.BlockSpec((_BM, dh), lambda i: (i, 0)),
        out_shape=jax.ShapeDtypeStruct((m, dh), jnp.float32),
    )(acc_a, acc_b, z1, dinv16, w2, b1_row)


def _tc3_body(aa_ref, ab_ref, z2_ref, dinv_ref, b2_ref, wf1_ref, bf1_ref,
              wf2_ref, bf2_ref, wt0_ref, bt0_ref, wt1_ref, bt1_ref,
              y0_ref, y1_ref, fx_ref):
    dv = dinv_ref[:, :1]
    h = jnp.maximum(dv * (aa_ref[...] + ab_ref[...] + z2_ref[...]) + b2_ref[...], 0.0)
    fx = jnp.maximum(
        jnp.dot(h, wf1_ref[...], preferred_element_type=jnp.float32) + bf1_ref[...], 0.0)
    fx_ref[...] = fx
    h2 = jnp.maximum(
        jnp.dot(fx, wf2_ref[...], preferred_element_type=jnp.float32) + bf2_ref[...], 0.0)
    y0_ref[...] = jnp.dot(h2, wt0_ref[...], preferred_element_type=jnp.float32) + bt0_ref[...]
    y1_ref[...] = jnp.dot(h2, wt1_ref[...], preferred_element_type=jnp.float32) + bt1_ref[...]


def _tc3(acc_a, acc_b, z2, dinv16, b2_row, wf1, bf1_row, wf2, bf2_row,
         wt0, bt0_row, wt1, bt1_row):
    m, dh = z2.shape
    dfc = wf1.shape[1]
    c0 = wt0.shape[1]
    c1 = wt1.shape[1]
    grid = (m // _BM,)
    return pl.pallas_call(
        _tc3_body,
        grid=grid,
        in_specs=[
            pl.BlockSpec((_BM, dh), lambda i: (i, 0)),
            pl.BlockSpec((_BM, dh), lambda i: (i, 0)),
            pl.BlockSpec((_BM, dh), lambda i: (i, 0)),
            pl.BlockSpec((_BM, 16), lambda i: (i, 0)),
            pl.BlockSpec((1, dh), lambda i: (0, 0)),
            pl.BlockSpec((dh, dfc), lambda i: (0, 0)),
            pl.BlockSpec((1, dfc), lambda i: (0, 0)),
            pl.BlockSpec((dfc, dfc), lambda i: (0, 0)),
            pl.BlockSpec((1, dfc), lambda i: (0, 0)),
            pl.BlockSpec((dfc, c0), lambda i: (0, 0)),
            pl.BlockSpec((1, c0), lambda i: (0, 0)),
            pl.BlockSpec((dfc, c1), lambda i: (0, 0)),
            pl.BlockSpec((1, c1), lambda i: (0, 0)),
        ],
        out_specs=[
            pl.BlockSpec((_BM, c0), lambda i: (i, 0)),
            pl.BlockSpec((_BM, c1), lambda i: (i, 0)),
            pl.BlockSpec((_BM, dfc), lambda i: (i, 0)),
        ],
        out_shape=[
            jax.ShapeDtypeStruct((m, c0), jnp.float32),
            jax.ShapeDtypeStruct((m, c1), jnp.float32),
            jax.ShapeDtypeStruct((m, dfc), jnp.float32),
        ],
    )(acc_a, acc_b, z2, dinv16, b2_row, wf1, bf1_row, wf2, bf2_row,
      wt0, bt0_row, wt1, bt1_row)


def kernel(x, edge_index, W1, b1, W2, b2, Wf1, bf1, Wf2, bf2, Wt0, bt0, Wt1, bt1):
    n, din = x.shape
    e = edge_index.shape[1]
    dh = W1.shape[1]

    src = edge_index[0]
    dst = edge_index[1]

    rpt = (n // NS) // 8 * 8
    rem = n - rpt * NS
    ones_w = jnp.ones((CH, 128), jnp.float32)
    zeros_d = jnp.zeros((rpt + rem, dh), jnp.float32)

    deg_kernel = _make_deg_kernel(n, e)
    agg_kernel = _make_agg_kernel(n, e, dh)

    xw1 = _tc0(x, W1)
    deg_a, deg_b = deg_kernel(dst, ones_w, zeros_d)
    z1, dinv16 = _tc1(xw1, deg_a, deg_b)
    acc_a1, acc_b1 = agg_kernel(z1, src, dst, zeros_d)
    z2 = _tc2(acc_a1, acc_b1, z1, dinv16, W2, b1.reshape(1, -1))
    acc_a2, acc_b2 = agg_kernel(z2, src, dst, zeros_d)
    y0, y1, fx = _tc3(
        acc_a2, acc_b2, z2, dinv16, b2.reshape(1, -1),
        Wf1, bf1.reshape(1, -1), Wf2, bf2.reshape(1, -1),
        Wt0, bt0.reshape(1, -1), Wt1, bt1.reshape(1, -1))
    return (y0, y1, fx)


# speedup vs baseline: 1.0228x; 1.0228x over previous
import functools

import jax
import jax.numpy as jnp
from jax import lax
from jax.experimental import pallas as pl
from jax.experimental.pallas import tpu as pltpu
from jax.experimental.pallas import tpu_sc as plsc

NC = 2
NS = 16
CH = 80
RING = 3


def _sc_mesh():
    return plsc.VectorSubcoreMesh(
        core_axis_name="c", subcore_axis_name="s", num_cores=NC, num_subcores=NS
    )


def _make_deg_kernel(n, e, w=128):
    tiles = NC * NS
    ept = e // tiles
    nch = ept // CH
    rpt = (n // NS) // 8 * 8
    rem = n - rpt * NS

    @functools.partial(
        pl.kernel,
        mesh=_sc_mesh(),
        out_type=[
            jax.ShapeDtypeStruct((n, w), jnp.float32),
            jax.ShapeDtypeStruct((n, w), jnp.float32),
        ],
        scratch_types=[
            pltpu.VMEM((CH,), jnp.int32),
            pltpu.VMEM((CH,), jnp.int32),
            pltpu.VMEM((CH, w), jnp.float32),
            pltpu.VMEM_SHARED((n, w), jnp.float32),
            pltpu.SemaphoreType.DMA,
            pltpu.SemaphoreType.DMA,
            pltpu.SemaphoreType.DMA,
            pltpu.SemaphoreType.DMA,
        ],
    )
    def deg_kernel(dst_hbm, ones_hbm, zeros_hbm, out_a, out_b,
                   dstv0, dstv1, onesv, acc, si0, si1, ss0, ss1):
        dstv = [dstv0, dstv1]
        si = [si0, si1]
        ss = [ss0, ss1]
        cid = lax.axis_index("c")
        sid = lax.axis_index("s")
        wid = cid * NS + sid
        row0 = sid * rpt
        pltpu.sync_copy(zeros_hbm.at[pl.ds(0, rpt)], acc.at[pl.ds(row0, rpt)])
        if rem:
            @pl.when(sid == NS - 1)
            def _():
                pltpu.sync_copy(zeros_hbm.at[pl.ds(rpt, rem)],
                                acc.at[pl.ds(NS * rpt, rem)])
        pltpu.sync_copy(ones_hbm, onesv)
        ebase = pl.multiple_of(wid * ept, 8)
        plsc.subcore_barrier()

        def slot(i, b):
            nb = 1 - b

            @pl.when(i >= 2)
            def _():
                pltpu.make_async_copy(onesv, acc.at[dstv[b]], ss[b]).wait()

            @pl.when(i < nch)
            def _():
                off = pl.multiple_of(ebase + i * CH, 8)
                pltpu.async_copy(dst_hbm.at[pl.ds(off, CH)], dstv[b], si[b])

            @pl.when(jnp.logical_and(i >= 1, i <= nch))
            def _():
                pltpu.make_async_copy(dst_hbm.at[pl.ds(ebase, CH)], dstv[nb],
                                      si[nb]).wait()
                pltpu.async_copy(onesv, acc.at[dstv[nb]], ss[nb], add=True)

        pairs = (nch + 2) // 2

        def pair(g, c):
            slot(2 * g, 0)
            slot(2 * g + 1, 1)
            return c

        lax.fori_loop(0, pairs, pair, 0)
        for i in range(max(0, 2 * pairs - 2), nch):
            pltpu.make_async_copy(onesv, acc.at[dstv[i % 2]], ss[i % 2]).wait()
        plsc.subcore_barrier()
        out = [out_a, out_b]
        for c in range(NC):
            @pl.when(cid == c)
            def _(o=out[c]):
                pltpu.sync_copy(acc.at[pl.ds(row0, rpt)], o.at[pl.ds(row0, rpt)])
                if rem:
                    @pl.when(sid == NS - 1)
                    def _():
                        pltpu.sync_copy(acc.at[pl.ds(NS * rpt, rem)],
                                        o.at[pl.ds(NS * rpt, rem)])

    return deg_kernel


def _make_agg_kernel(n, e, d):
    tiles = NC * NS
    ept = e // tiles
    nch = ept // CH
    rpt = (n // NS) // 8 * 8
    rem = n - rpt * NS
    R = RING

    @functools.partial(
        pl.kernel,
        mesh=_sc_mesh(),
        out_type=[
            jax.ShapeDtypeStruct((n, d), jnp.float32),
            jax.ShapeDtypeStruct((n, d), jnp.float32),
        ],
        scratch_types=[pltpu.VMEM((ept,), jnp.int32)]
        + [pltpu.VMEM((CH,), jnp.int32)] * R
        + [pltpu.VMEM((CH, d), jnp.float32)] * R
        + [pltpu.VMEM_SHARED((n, d), jnp.float32)]
        + [pltpu.SemaphoreType.DMA] * (3 * R),
    )
    def agg_kernel(z_hbm, src_hbm, dst_hbm, zeros_hbm, out_a, out_b, *scr):
        srcall = scr[0]
        dstv = list(scr[1:1 + R])
        rows = list(scr[1 + R:1 + 2 * R])
        acc = scr[1 + 2 * R]
        sems = scr[2 + 2 * R:]
        sg = list(sems[0:R])
        si = list(sems[R:2 * R])
        ss = list(sems[2 * R:3 * R])
        cid = lax.axis_index("c")
        sid = lax.axis_index("s")
        wid = cid * NS + sid
        row0 = sid * rpt
        pltpu.sync_copy(zeros_hbm.at[pl.ds(0, rpt)], acc.at[pl.ds(row0, rpt)])
        if rem:
            @pl.when(sid == NS - 1)
            def _():
                pltpu.sync_copy(zeros_hbm.at[pl.ds(rpt, rem)],
                                acc.at[pl.ds(NS * rpt, rem)])
        ebase = pl.multiple_of(wid * ept, 8)
        pltpu.sync_copy(src_hbm.at[pl.ds(ebase, ept)], srcall)
        plsc.subcore_barrier()

        def gidx(i):
            return srcall.at[pl.ds(pl.multiple_of(i * CH, 8), CH)]

        D = 2

        def slot(i, b):
            nb = (b - D) % R

            @pl.when(jnp.logical_and(i >= R, i < nch + R))
            def _():
                pltpu.make_async_copy(rows[b], acc.at[dstv[b]], ss[b]).wait()

            @pl.when(i < nch)
            def _():
                pltpu.async_copy(z_hbm.at[gidx(i)], rows[b], sg[b])
                off = pl.multiple_of(ebase + i * CH, 8)
                pltpu.async_copy(dst_hbm.at[pl.ds(off, CH)], dstv[b], si[b])

            @pl.when(jnp.logical_and(i >= D, i <= nch + D - 1))
            def _():
                pltpu.make_async_copy(z_hbm.at[gidx(0)], rows[nb], sg[nb]).wait()
                pltpu.make_async_copy(dst_hbm.at[pl.ds(ebase, CH)], dstv[nb],
                                      si[nb]).wait()
                pltpu.async_copy(rows[nb], acc.at[dstv[nb]], ss[nb], add=True)

        ngrp = (nch + D + R - 1) // R
        s_last = ngrp * R - 1

        def grp(g, c):
            for b in range(R):
                slot(R * g + b, b)
            return c

        lax.fori_loop(0, ngrp, grp, 0)
        for i in range(max(0, s_last - R + 1), nch):
            pltpu.make_async_copy(rows[i % R], acc.at[dstv[i % R]],
                                  ss[i % R]).wait()
        plsc.subcore_barrier()
        out = [out_a, out_b]
        for c in range(NC):
            @pl.when(cid == c)
            def _(o=out[c]):
                pltpu.sync_copy(acc.at[pl.ds(row0, rpt)], o.at[pl.ds(row0, rpt)])
                if rem:
                    @pl.when(sid == NS - 1)
                    def _():
                        pltpu.sync_copy(acc.at[pl.ds(NS * rpt, rem)],
                                        o.at[pl.ds(NS * rpt, rem)])

    return agg_kernel


_BM = 1000


def _tc0_body(x_ref, w_ref, xw_ref):
    xw_ref[...] = jnp.dot(x_ref[...], w_ref[...],
                          preferred_element_type=jnp.float32)


def _tc0(x, w1):
    m, din = x.shape
    dh = w1.shape[1]
    grid = (m // _BM,)
    return pl.pallas_call(
        _tc0_body,
        grid=grid,
        in_specs=[
            pl.BlockSpec((_BM, din), lambda i: (i, 0)),
            pl.BlockSpec((din, dh), lambda i: (0, 0)),
        ],
        out_specs=pl.BlockSpec((_BM, dh), lambda i: (i, 0)),
        out_shape=jax.ShapeDtypeStruct((m, dh), jnp.float32),
    )(x, w1)


def _tc1_body(xw_ref, da_ref, db_ref, z_ref, dinv_ref):
    deg = da_ref[:, :1] + db_ref[:, :1] + 1.0
    dinv = lax.rsqrt(deg)
    z_ref[...] = dinv * xw_ref[...]
    dinv_ref[...] = jnp.broadcast_to(dinv, (dinv.shape[0], 16))


def _tc1(xw, deg_a, deg_b):
    m, dh = xw.shape
    grid = (m // _BM,)
    return pl.pallas_call(
        _tc1_body,
        grid=grid,
        in_specs=[
            pl.BlockSpec((_BM, dh), lambda i: (i, 0)),
            pl.BlockSpec((_BM, 128), lambda i: (i, 0)),
            pl.BlockSpec((_BM, 128), lambda i: (i, 0)),
        ],
        out_specs=[
            pl.BlockSpec((_BM, dh), lambda i: (i, 0)),
            pl.BlockSpec((_BM, 16), lambda i: (i, 0)),
        ],
        out_shape=[
            jax.ShapeDtypeStruct((m, dh), jnp.float32),
            jax.ShapeDtypeStruct((m, 16), jnp.float32),
        ],
    )(xw, deg_a, deg_b)


def _tc2_body(aa_ref, ab_ref, z1_ref, dinv_ref, w2_ref, b1_ref, z2_ref):
    dv = dinv_ref[:, :1]
    h = jnp.maximum(dv * (aa_ref[...] + ab_ref[...] + z1_ref[...]) + b1_ref[...], 0.0)
    z2_ref[...] = dv * jnp.dot(h, w2_ref[...], preferred_element_type=jnp.float32)


def _tc2(acc_a, acc_b, z1, dinv16, w2, b1_row):
    m, dh = z1.shape
    grid = (m // _BM,)
    return pl.pallas_call(
        _tc2_body,
        grid=grid,
        in_specs=[
            pl.BlockSpec((_BM, dh), lambda i: (i, 0)),
            pl.BlockSpec((_BM, dh), lambda i: (i, 0)),
            pl.BlockSpec((_BM, dh), lambda i: (i, 0)),
            pl.BlockSpec((_BM, 16), lambda i: (i, 0)),
            pl.BlockSpec((dh, dh), lambda i: (0, 0)),
            pl.BlockSpec((1, dh), lambda i: (0, 0)),
        ],
        out_specs=pl.BlockSpec((_BM, dh), lambda i: (i, 0)),
        out_shape=jax.ShapeDtypeStruct((m, dh), jnp.float32),
    )(acc_a, acc_b, z1, dinv16, w2, b1_row)


def _tc3_body(aa_ref, ab_ref, z2_ref, dinv_ref, b2_ref, wf1_ref, bf1_ref,
              wf2_ref, bf2_ref, wt0_ref, bt0_ref, wt1_ref, bt1_ref,
              y0_ref, y1_ref, fx_ref):
    dv = dinv_ref[:, :1]
    h = jnp.maximum(dv * (aa_ref[...] + ab_ref[...] + z2_ref[...]) + b2_ref[...], 0.0)
    fx = jnp.maximum(
        jnp.dot(h, wf1_ref[...], preferred_element_type=jnp.float32) + bf1_ref[...], 0.0)
    fx_ref[...] = fx
    h2 = jnp.maximum(
        jnp.dot(fx, wf2_ref[...], preferred_element_type=jnp.float32) + bf2_ref[...], 0.0)
    y0_ref[...] = jnp.dot(h2, wt0_ref[...], preferred_element_type=jnp.float32) + bt0_ref[...]
    y1_ref[...] = jnp.dot(h2, wt1_ref[...], preferred_element_type=jnp.float32) + bt1_ref[...]


def _tc3(acc_a, acc_b, z2, dinv16, b2_row, wf1, bf1_row, wf2, bf2_row,
         wt0, bt0_row, wt1, bt1_row):
    m, dh = z2.shape
    dfc = wf1.shape[1]
    c0 = wt0.shape[1]
    c1 = wt1.shape[1]
    grid = (m // _BM,)
    return pl.pallas_call(
        _tc3_body,
        grid=grid,
        in_specs=[
            pl.BlockSpec((_BM, dh), lambda i: (i, 0)),
            pl.BlockSpec((_BM, dh), lambda i: (i, 0)),
            pl.BlockSpec((_BM, dh), lambda i: (i, 0)),
            pl.BlockSpec((_BM, 16), lambda i: (i, 0)),
            pl.BlockSpec((1, dh), lambda i: (0, 0)),
            pl.BlockSpec((dh, dfc), lambda i: (0, 0)),
            pl.BlockSpec((1, dfc), lambda i: (0, 0)),
            pl.BlockSpec((dfc, dfc), lambda i: (0, 0)),
            pl.BlockSpec((1, dfc), lambda i: (0, 0)),
            pl.BlockSpec((dfc, c0), lambda i: (0, 0)),
            pl.BlockSpec((1, c0), lambda i: (0, 0)),
            pl.BlockSpec((dfc, c1), lambda i: (0, 0)),
            pl.BlockSpec((1, c1), lambda i: (0, 0)),
        ],
        out_specs=[
            pl.BlockSpec((_BM, c0), lambda i: (i, 0)),
            pl.BlockSpec((_BM, c1), lambda i: (i, 0)),
            pl.BlockSpec((_BM, dfc), lambda i: (i, 0)),
        ],
        out_shape=[
            jax.ShapeDtypeStruct((m, c0), jnp.float32),
            jax.ShapeDtypeStruct((m, c1), jnp.float32),
            jax.ShapeDtypeStruct((m, dfc), jnp.float32),
        ],
    )(acc_a, acc_b, z2, dinv16, b2_row, wf1, bf1_row, wf2, bf2_row,
      wt0, bt0_row, wt1, bt1_row)


def kernel(x, edge_index, W1, b1, W2, b2, Wf1, bf1, Wf2, bf2, Wt0, bt0, Wt1, bt1):
    n, din = x.shape
    e = edge_index.shape[1]
    dh = W1.shape[1]

    src = edge_index[0]
    dst = edge_index[1]

    rpt = (n // NS) // 8 * 8
    rem = n - rpt * NS
    ones_w = jnp.ones((CH, 128), jnp.float32)
    zeros_d = jnp.zeros((rpt + rem, dh), jnp.float32)

    deg_kernel = _make_deg_kernel(n, e)
    agg_kernel = _make_agg_kernel(n, e, dh)

    xw1 = _tc0(x, W1)
    deg_a, deg_b = deg_kernel(dst, ones_w, zeros_d)
    z1, dinv16 = _tc1(xw1, deg_a, deg_b)
    acc_a1, acc_b1 = agg_kernel(z1, src, dst, zeros_d)
    z2 = _tc2(acc_a1, acc_b1, z1, dinv16, W2, b1.reshape(1, -1))
    acc_a2, acc_b2 = agg_kernel(z2, src, dst, zeros_d)
    y0, y1, fx = _tc3(
        acc_a2, acc_b2, z2, dinv16, b2.reshape(1, -1),
        Wf1, bf1.reshape(1, -1), Wf2, bf2.reshape(1, -1),
        Wt0, bt0.reshape(1, -1), Wt1, bt1.reshape(1, -1))
    return (y0, y1, fx)


# speedup vs baseline: 1.0403x; 1.0171x over previous
import functools

import jax
import jax.numpy as jnp
from jax import lax
from jax.experimental import pallas as pl
from jax.experimental.pallas import tpu as pltpu
from jax.experimental.pallas import tpu_sc as plsc

NC = 2
NS = 16
CH = 80
RING = 3


def _sc_mesh():
    return plsc.VectorSubcoreMesh(
        core_axis_name="c", subcore_axis_name="s", num_cores=NC, num_subcores=NS
    )


def _make_deg_kernel(n, e, w=128):
    tiles = NC * NS
    ept = e // tiles
    nch = ept // CH
    rpt = (n // NS) // 8 * 8
    rem = n - rpt * NS

    @functools.partial(
        pl.kernel,
        mesh=_sc_mesh(),
        out_type=[
            jax.ShapeDtypeStruct((n, w), jnp.float32),
            jax.ShapeDtypeStruct((n, w), jnp.float32),
        ],
        scratch_types=[
            pltpu.VMEM((CH,), jnp.int32),
            pltpu.VMEM((CH,), jnp.int32),
            pltpu.VMEM((CH, w), jnp.float32),
            pltpu.VMEM_SHARED((n, w), jnp.float32),
            pltpu.SemaphoreType.DMA,
            pltpu.SemaphoreType.DMA,
            pltpu.SemaphoreType.DMA,
            pltpu.SemaphoreType.DMA,
        ],
    )
    def deg_kernel(dst_hbm, ones_hbm, zeros_hbm, out_a, out_b,
                   dstv0, dstv1, onesv, acc, si0, si1, ss0, ss1):
        dstv = [dstv0, dstv1]
        si = [si0, si1]
        ss = [ss0, ss1]
        cid = lax.axis_index("c")
        sid = lax.axis_index("s")
        wid = cid * NS + sid
        row0 = sid * rpt
        pltpu.sync_copy(zeros_hbm.at[pl.ds(0, rpt)], acc.at[pl.ds(row0, rpt)])
        if rem:
            @pl.when(sid == NS - 1)
            def _():
                pltpu.sync_copy(zeros_hbm.at[pl.ds(rpt, rem)],
                                acc.at[pl.ds(NS * rpt, rem)])
        pltpu.sync_copy(ones_hbm, onesv)
        ebase = pl.multiple_of(wid * ept, 8)
        plsc.subcore_barrier()

        def slot(i, b):
            nb = 1 - b

            @pl.when(i >= 2)
            def _():
                pltpu.make_async_copy(onesv, acc.at[dstv[b]], ss[b]).wait()

            @pl.when(i < nch)
            def _():
                off = pl.multiple_of(ebase + i * CH, 8)
                pltpu.async_copy(dst_hbm.at[pl.ds(off, CH)], dstv[b], si[b])

            @pl.when(jnp.logical_and(i >= 1, i <= nch))
            def _():
                pltpu.make_async_copy(dst_hbm.at[pl.ds(ebase, CH)], dstv[nb],
                                      si[nb]).wait()
                pltpu.async_copy(onesv, acc.at[dstv[nb]], ss[nb], add=True)

        pairs = (nch + 2) // 2

        def pair(g, c):
            slot(2 * g, 0)
            slot(2 * g + 1, 1)
            return c

        lax.fori_loop(0, pairs, pair, 0)
        for i in range(max(0, 2 * pairs - 2), nch):
            pltpu.make_async_copy(onesv, acc.at[dstv[i % 2]], ss[i % 2]).wait()
        plsc.subcore_barrier()
        out = [out_a, out_b]
        for c in range(NC):
            @pl.when(cid == c)
            def _(o=out[c]):
                pltpu.sync_copy(acc.at[pl.ds(row0, rpt)], o.at[pl.ds(row0, rpt)])
                if rem:
                    @pl.when(sid == NS - 1)
                    def _():
                        pltpu.sync_copy(acc.at[pl.ds(NS * rpt, rem)],
                                        o.at[pl.ds(NS * rpt, rem)])

    return deg_kernel


def _make_agg_kernel(n, e, d):
    tiles = NC * NS
    ept = e // tiles
    nch = ept // CH
    rpt = (n // NS) // 8 * 8
    rem = n - rpt * NS
    R = RING

    @functools.partial(
        pl.kernel,
        mesh=_sc_mesh(),
        out_type=[
            jax.ShapeDtypeStruct((n, d), jnp.float32),
            jax.ShapeDtypeStruct((n, d), jnp.float32),
        ],
        scratch_types=[pltpu.VMEM((ept,), jnp.int32)]
        + [pltpu.VMEM((CH,), jnp.int32)] * R
        + [pltpu.VMEM((CH, d), jnp.float32)] * R
        + [pltpu.VMEM_SHARED((n, d), jnp.float32)]
        + [pltpu.SemaphoreType.DMA] * (3 * R),
    )
    def agg_kernel(z_hbm, src_hbm, dst_hbm, zeros_hbm, out_a, out_b, *scr):
        srcall = scr[0]
        dstv = list(scr[1:1 + R])
        rows = list(scr[1 + R:1 + 2 * R])
        acc = scr[1 + 2 * R]
        sems = scr[2 + 2 * R:]
        sg = list(sems[0:R])
        si = list(sems[R:2 * R])
        ss = list(sems[2 * R:3 * R])
        cid = lax.axis_index("c")
        sid = lax.axis_index("s")
        wid = cid * NS + sid
        row0 = sid * rpt
        pltpu.sync_copy(zeros_hbm.at[pl.ds(0, rpt)], acc.at[pl.ds(row0, rpt)])
        if rem:
            @pl.when(sid == NS - 1)
            def _():
                pltpu.sync_copy(zeros_hbm.at[pl.ds(rpt, rem)],
                                acc.at[pl.ds(NS * rpt, rem)])
        ebase = pl.multiple_of(wid * ept, 8)
        pltpu.sync_copy(src_hbm.at[pl.ds(ebase, ept)], srcall)
        plsc.subcore_barrier()

        def gidx(i):
            return srcall.at[pl.ds(pl.multiple_of(i * CH, 8), CH)]

        D = 2

        def slot(i, b):
            nb = (b - D) % R

            @pl.when(jnp.logical_and(i >= R, i < nch + R))
            def _():
                pltpu.make_async_copy(rows[b], acc.at[dstv[b]], ss[b]).wait()

            @pl.when(i < nch)
            def _():
                pltpu.async_copy(z_hbm.at[gidx(i)], rows[b], sg[b])
                off = pl.multiple_of(ebase + i * CH, 8)
                pltpu.async_copy(dst_hbm.at[pl.ds(off, CH)], dstv[b], si[b])

            @pl.when(jnp.logical_and(i >= D, i <= nch + D - 1))
            def _():
                pltpu.make_async_copy(z_hbm.at[gidx(0)], rows[nb], sg[nb]).wait()
                pltpu.make_async_copy(dst_hbm.at[pl.ds(ebase, CH)], dstv[nb],
                                      si[nb]).wait()
                pltpu.async_copy(rows[nb], acc.at[dstv[nb]], ss[nb], add=True)

        ngrp = (nch + D + R - 1) // R
        s_last = ngrp * R - 1

        def grp(g, c):
            for b in range(R):
                slot(R * g + b, b)
            return c

        lax.fori_loop(0, ngrp, grp, 0)
        for i in range(max(0, s_last - R + 1), nch):
            pltpu.make_async_copy(rows[i % R], acc.at[dstv[i % R]],
                                  ss[i % R]).wait()
        plsc.subcore_barrier()
        out = [out_a, out_b]
        for c in range(NC):
            @pl.when(cid == c)
            def _(o=out[c]):
                pltpu.sync_copy(acc.at[pl.ds(row0, rpt)], o.at[pl.ds(row0, rpt)])
                if rem:
                    @pl.when(sid == NS - 1)
                    def _():
                        pltpu.sync_copy(acc.at[pl.ds(NS * rpt, rem)],
                                        o.at[pl.ds(NS * rpt, rem)])

    return agg_kernel


_BM = 2000


def _tc0_body(x_ref, w_ref, xw_ref):
    xw_ref[...] = jnp.dot(x_ref[...], w_ref[...],
                          preferred_element_type=jnp.float32)


def _tc0(x, w1):
    m, din = x.shape
    dh = w1.shape[1]
    grid = (m // _BM,)
    return pl.pallas_call(
        _tc0_body,
        grid=grid,
        in_specs=[
            pl.BlockSpec((_BM, din), lambda i: (i, 0)),
            pl.BlockSpec((din, dh), lambda i: (0, 0)),
        ],
        out_specs=pl.BlockSpec((_BM, dh), lambda i: (i, 0)),
        out_shape=jax.ShapeDtypeStruct((m, dh), jnp.float32),
    )(x, w1)


def _tc1_body(xw_ref, da_ref, db_ref, z_ref, dinv_ref):
    deg = da_ref[:, :1] + db_ref[:, :1] + 1.0
    dinv = lax.rsqrt(deg)
    z_ref[...] = dinv * xw_ref[...]
    dinv_ref[...] = jnp.broadcast_to(dinv, (dinv.shape[0], 16))


def _tc1(xw, deg_a, deg_b):
    m, dh = xw.shape
    grid = (m // _BM,)
    return pl.pallas_call(
        _tc1_body,
        grid=grid,
        in_specs=[
            pl.BlockSpec((_BM, dh), lambda i: (i, 0)),
            pl.BlockSpec((_BM, 128), lambda i: (i, 0)),
            pl.BlockSpec((_BM, 128), lambda i: (i, 0)),
        ],
        out_specs=[
            pl.BlockSpec((_BM, dh), lambda i: (i, 0)),
            pl.BlockSpec((_BM, 16), lambda i: (i, 0)),
        ],
        out_shape=[
            jax.ShapeDtypeStruct((m, dh), jnp.float32),
            jax.ShapeDtypeStruct((m, 16), jnp.float32),
        ],
    )(xw, deg_a, deg_b)


def _tc2_body(aa_ref, ab_ref, z1_ref, dinv_ref, w2_ref, b1_ref, z2_ref):
    dv = dinv_ref[:, :1]
    h = jnp.maximum(dv * (aa_ref[...] + ab_ref[...] + z1_ref[...]) + b1_ref[...], 0.0)
    z2_ref[...] = dv * jnp.dot(h, w2_ref[...], preferred_element_type=jnp.float32)


def _tc2(acc_a, acc_b, z1, dinv16, w2, b1_row):
    m, dh = z1.shape
    grid = (m // _BM,)
    return pl.pallas_call(
        _tc2_body,
        grid=grid,
        in_specs=[
            pl.BlockSpec((_BM, dh), lambda i: (i, 0)),
            pl.BlockSpec((_BM, dh), lambda i: (i, 0)),
            pl.BlockSpec((_BM, dh), lambda i: (i, 0)),
            pl.BlockSpec((_BM, 16), lambda i: (i, 0)),
            pl.BlockSpec((dh, dh), lambda i: (0, 0)),
            pl.BlockSpec((1, dh), lambda i: (0, 0)),
        ],
        out_specs=pl.BlockSpec((_BM, dh), lambda i: (i, 0)),
        out_shape=jax.ShapeDtypeStruct((m, dh), jnp.float32),
    )(acc_a, acc_b, z1, dinv16, w2, b1_row)


def _tc3_body(aa_ref, ab_ref, z2_ref, dinv_ref, b2_ref, wf1_ref, bf1_ref,
              wf2_ref, bf2_ref, wt0_ref, bt0_ref, wt1_ref, bt1_ref,
              y0_ref, y1_ref, fx_ref):
    dv = dinv_ref[:, :1]
    h = jnp.maximum(dv * (aa_ref[...] + ab_ref[...] + z2_ref[...]) + b2_ref[...], 0.0)
    fx = jnp.maximum(
        jnp.dot(h, wf1_ref[...], preferred_element_type=jnp.float32) + bf1_ref[...], 0.0)
    fx_ref[...] = fx
    h2 = jnp.maximum(
        jnp.dot(fx, wf2_ref[...], preferred_element_type=jnp.float32) + bf2_ref[...], 0.0)
    y0_ref[...] = jnp.dot(h2, wt0_ref[...], preferred_element_type=jnp.float32) + bt0_ref[...]
    y1_ref[...] = jnp.dot(h2, wt1_ref[...], preferred_element_type=jnp.float32) + bt1_ref[...]


def _tc3(acc_a, acc_b, z2, dinv16, b2_row, wf1, bf1_row, wf2, bf2_row,
         wt0, bt0_row, wt1, bt1_row):
    m, dh = z2.shape
    dfc = wf1.shape[1]
    c0 = wt0.shape[1]
    c1 = wt1.shape[1]
    grid = (m // _BM,)
    return pl.pallas_call(
        _tc3_body,
        grid=grid,
        in_specs=[
            pl.BlockSpec((_BM, dh), lambda i: (i, 0)),
            pl.BlockSpec((_BM, dh), lambda i: (i, 0)),
            pl.BlockSpec((_BM, dh), lambda i: (i, 0)),
            pl.BlockSpec((_BM, 16), lambda i: (i, 0)),
            pl.BlockSpec((1, dh), lambda i: (0, 0)),
            pl.BlockSpec((dh, dfc), lambda i: (0, 0)),
            pl.BlockSpec((1, dfc), lambda i: (0, 0)),
            pl.BlockSpec((dfc, dfc), lambda i: (0, 0)),
            pl.BlockSpec((1, dfc), lambda i: (0, 0)),
            pl.BlockSpec((dfc, c0), lambda i: (0, 0)),
            pl.BlockSpec((1, c0), lambda i: (0, 0)),
            pl.BlockSpec((dfc, c1), lambda i: (0, 0)),
            pl.BlockSpec((1, c1), lambda i: (0, 0)),
        ],
        out_specs=[
            pl.BlockSpec((_BM, c0), lambda i: (i, 0)),
            pl.BlockSpec((_BM, c1), lambda i: (i, 0)),
            pl.BlockSpec((_BM, dfc), lambda i: (i, 0)),
        ],
        out_shape=[
            jax.ShapeDtypeStruct((m, c0), jnp.float32),
            jax.ShapeDtypeStruct((m, c1), jnp.float32),
            jax.ShapeDtypeStruct((m, dfc), jnp.float32),
        ],
    )(acc_a, acc_b, z2, dinv16, b2_row, wf1, bf1_row, wf2, bf2_row,
      wt0, bt0_row, wt1, bt1_row)


def kernel(x, edge_index, W1, b1, W2, b2, Wf1, bf1, Wf2, bf2, Wt0, bt0, Wt1, bt1):
    n, din = x.shape
    e = edge_index.shape[1]
    dh = W1.shape[1]

    src = edge_index[0]
    dst = edge_index[1]

    rpt = (n // NS) // 8 * 8
    rem = n - rpt * NS
    ones_w = jnp.ones((CH, 128), jnp.float32)
    zeros_d = jnp.zeros((rpt + rem, dh), jnp.float32)

    deg_kernel = _make_deg_kernel(n, e)
    agg_kernel = _make_agg_kernel(n, e, dh)

    xw1 = _tc0(x, W1)
    deg_a, deg_b = deg_kernel(dst, ones_w, zeros_d)
    z1, dinv16 = _tc1(xw1, deg_a, deg_b)
    acc_a1, acc_b1 = agg_kernel(z1, src, dst, zeros_d)
    z2 = _tc2(acc_a1, acc_b1, z1, dinv16, W2, b1.reshape(1, -1))
    acc_a2, acc_b2 = agg_kernel(z2, src, dst, zeros_d)
    y0, y1, fx = _tc3(
        acc_a2, acc_b2, z2, dinv16, b2.reshape(1, -1),
        Wf1, bf1.reshape(1, -1), Wf2, bf2.reshape(1, -1),
        Wt0, bt0.reshape(1, -1), Wt1, bt1.reshape(1, -1))
    return (y0, y1, fx)
